# baseline (speedup 1.0000x reference)
import functools

import jax
import jax.numpy as jnp
from jax import lax
from jax.experimental import pallas as pl
from jax.experimental.pallas import tpu as pltpu
from jax.experimental.pallas import tpu_sc as plsc

VOCAB = 1000
N_EMBD = 32
T = 8
BATCH = 16384
NROW = BATCH * T
NC = 2
NS = 16
NW = NC * NS
PER_W = NROW // NW
C = 32
NCHUNK = PER_W // C


def _comb_body(tok_ref, pos_ref, wt_ref, b_ref, out_ref):
    t = pl.program_id(0)
    x = tok_ref[...] + pos_ref[pl.ds(t, 1), :]
    y = jnp.dot(x, wt_ref[...], preferred_element_type=jnp.float32)
    out_ref[...] = (y + b_ref[...])[None]


def _build_comb(tok_table, pos_table, Wt, b2):
    return pl.pallas_call(
        _comb_body,
        grid=(T,),
        in_specs=[
            pl.BlockSpec((VOCAB, N_EMBD), lambda t: (0, 0)),
            pl.BlockSpec((T, N_EMBD), lambda t: (0, 0)),
            pl.BlockSpec((N_EMBD, VOCAB), lambda t: (0, 0)),
            pl.BlockSpec((1, VOCAB), lambda t: (0, 0)),
        ],
        out_specs=pl.BlockSpec((1, VOCAB, VOCAB), lambda t: (t, 0, 0)),
        out_shape=jax.ShapeDtypeStruct((T, VOCAB, VOCAB), jnp.float32),
    )(tok_table, pos_table, Wt, b2)


@functools.cache
def _make_sc_gather():
    mesh = plsc.VectorSubcoreMesh(core_axis_name="c", subcore_axis_name="s")
    return _sc_kernel_def(mesh)


def _sc_kernel_def(mesh):
    return functools.partial(
        pl.kernel,
        out_type=jax.ShapeDtypeStruct((NROW, VOCAB), jnp.float32),
        mesh=mesh,
        compiler_params=pltpu.CompilerParams(use_tc_tiling_on_sc=False),
        scratch_types=[
        pltpu.VMEM((PER_W,), jnp.int32),
        pltpu.VMEM((NCHUNK, C), jnp.int32),
        pltpu.VMEM((C, VOCAB), jnp.float32),
        pltpu.VMEM((C, VOCAB), jnp.float32),
        pltpu.SemaphoreType.DMA,
        pltpu.SemaphoreType.DMA,
        pltpu.SemaphoreType.DMA,
        pltpu.SemaphoreType.DMA,
    ],
    )(_sc_body)


def _sc_body(idx_hbm, comb_hbm, out_hbm,
               idx_v, cidx_v, buf_a, buf_b, gs_a, gs_b, ss_a, ss_b):
    wid = lax.axis_index("s") * NC + lax.axis_index("c")
    base = wid * PER_W
    pltpu.sync_copy(idx_hbm.at[pl.ds(base, PER_W)], idx_v)

    tvec = (lax.iota(jnp.int32, 16) % T) * VOCAB

    def cbody(k, carry):
        for h in range(2):
            v = idx_v[pl.ds(k * C + h * 16, 16)]
            cidx_v[k, pl.ds(h * 16, 16)] = v + tvec
        return carry

    lax.fori_loop(0, NCHUNK, cbody, 0)

    bufs = (buf_a, buf_b)
    gsems = (gs_a, gs_b)
    ssems = (ss_a, ss_b)

    def g_start(k, p):
        pltpu.async_copy(comb_hbm.at[cidx_v.at[k]], bufs[p], gsems[p])

    def g_wait(k, p):
        pltpu.make_async_copy(comb_hbm.at[cidx_v.at[k]], bufs[p], gsems[p]).wait()

    def s_start(k, p):
        pltpu.async_copy(bufs[p], out_hbm.at[pl.ds(base + k * C, C), :], ssems[p])

    def s_wait(k, p):
        pltpu.make_async_copy(
            bufs[p], out_hbm.at[pl.ds(base + k * C, C), :], ssems[p]).wait()

    g_start(0, 0)
    g_start(1, 1)

    def lbody(k2, carry):
        for p in range(2):
            k = k2 * 2 + p
            g_wait(k, p)
            s_start(k, p)
            s_wait(k, p)

            @pl.when(k2 < NCHUNK // 2 - 1)
            def _():
                g_start(k + 2, p)
        return carry

    lax.fori_loop(0, NCHUNK // 2, lbody, 0)


def kernel(idx, tok_table, pos_table, W, b):
    comb = _build_comb(tok_table, pos_table,
                       W.T, b.reshape(1, VOCAB)).reshape(T * VOCAB, VOCAB)
    out = _make_sc_gather()(idx.reshape(NROW), comb)
    return out.reshape(BATCH, T, VOCAB)


# speedup vs baseline: 2.9987x; 2.9987x over previous
import functools

import jax
import jax.numpy as jnp
from jax import lax
from jax.experimental import pallas as pl
from jax.experimental.pallas import tpu as pltpu
from jax.experimental.pallas import tpu_sc as plsc

VOCAB = 1000
N_EMBD = 32
T = 8
BATCH = 16384
NROW = BATCH * T
NC = 2
NS = 16
NW = NC * NS
PER_W = NROW // NW
C = 128
NCHUNK = PER_W // C

BBLK = 2048
NBLK = BATCH // BBLK


@functools.cache
def _make_sc_gather():
    mesh = plsc.VectorSubcoreMesh(core_axis_name="c", subcore_axis_name="s")
    return functools.partial(
        pl.kernel,
        out_type=jax.ShapeDtypeStruct((NROW, N_EMBD), jnp.float32),
        mesh=mesh,
        compiler_params=pltpu.CompilerParams(use_tc_tiling_on_sc=False),
        scratch_types=[
            pltpu.VMEM((NCHUNK, C), jnp.int32),
            pltpu.VMEM((C, N_EMBD), jnp.float32),
            pltpu.VMEM((C, N_EMBD), jnp.float32),
            pltpu.SemaphoreType.DMA,
            pltpu.SemaphoreType.DMA,
            pltpu.SemaphoreType.DMA,
            pltpu.SemaphoreType.DMA,
        ],
    )(_sc_body)


def _sc_body(idx_hbm, tok_hbm, out_hbm, idx_v, buf_a, buf_b, gs_a, gs_b, ss_a, ss_b):
    wid = lax.axis_index("s") * NC + lax.axis_index("c")
    base = wid * PER_W
    pltpu.sync_copy(idx_hbm.at[pl.ds(wid * NCHUNK, NCHUNK), :], idx_v)

    bufs = (buf_a, buf_b)
    gsems = (gs_a, gs_b)
    ssems = (ss_a, ss_b)

    def g_start(k, p):
        pltpu.async_copy(tok_hbm.at[idx_v.at[k]], bufs[p], gsems[p])

    def g_wait(k, p):
        pltpu.make_async_copy(tok_hbm.at[idx_v.at[k]], bufs[p], gsems[p]).wait()

    def s_start(k, p):
        pltpu.async_copy(bufs[p], out_hbm.at[pl.ds(base + k * C, C), :], ssems[p])

    def s_wait(k, p):
        pltpu.make_async_copy(
            bufs[p], out_hbm.at[pl.ds(base + k * C, C), :], ssems[p]).wait()

    g_start(0, 0)
    g_start(1, 1)

    def lbody(k2, carry):
        for p in range(2):
            k = k2 * 2 + p
            g_wait(k, p)
            s_start(k, p)
            s_wait(k, p)

            @pl.when(k2 < NCHUNK // 2 - 1)
            def _():
                g_start(k + 2, p)
        return carry

    lax.fori_loop(0, NCHUNK // 2, lbody, 0)


def _proj_body(g_ref, pos_ref, w_ref, b_ref, out_ref):
    t = pl.program_id(0)
    x = g_ref[:, 0, 0, :] + pos_ref[pl.ds(t, 1), :]
    y = lax.dot_general(w_ref[...], x.astype(jnp.bfloat16),
                        (((1,), (1,)), ((), ())),
                        preferred_element_type=jnp.float32)
    out_ref[...] = (y + b_ref[...])[None]


def _project(g4, pos_table, w_bf, b_col):
    return pl.pallas_call(
        _proj_body,
        grid=(T, NBLK),
        in_specs=[
            pl.BlockSpec((BBLK, 1, 1, N_EMBD), lambda t, k: (k, t, 0, 0)),
            pl.BlockSpec((T, N_EMBD), lambda t, k: (0, 0)),
            pl.BlockSpec((VOCAB, N_EMBD), lambda t, k: (0, 0)),
            pl.BlockSpec((VOCAB, 1), lambda t, k: (0, 0)),
        ],
        out_specs=pl.BlockSpec((1, VOCAB, BBLK), lambda t, k: (t, 0, k)),
        out_shape=jax.ShapeDtypeStruct((T, VOCAB, BATCH), jnp.float32),
    )(g4, pos_table, w_bf, b_col)


def kernel(idx, tok_table, pos_table, W, b):
    g = _make_sc_gather()(idx.reshape(NROW // C, C), tok_table)
    g4 = g.reshape(BATCH, T, 1, N_EMBD)
    out3 = _project(g4, pos_table, W.astype(jnp.bfloat16), b.reshape(VOCAB, 1))
    return jnp.transpose(out3, (2, 0, 1))


# speedup vs baseline: 5.6408x; 1.8810x over previous
import functools

import jax
import jax.numpy as jnp
from jax import lax
from jax.experimental import pallas as pl
from jax.experimental.pallas import tpu as pltpu
from jax.experimental.pallas import tpu_sc as plsc

VOCAB = 1000
N_EMBD = 32
T = 8
BATCH = 16384
NROW = BATCH * T
NC = 2
NS = 16
NW = NC * NS
PER_W = NROW // NW
C = 128
NCHUNK = PER_W // C

BBLK = 512
NBLK = BATCH // BBLK


@functools.cache
def _make_sc_gather():
    mesh = plsc.VectorSubcoreMesh(core_axis_name="c", subcore_axis_name="s")
    return functools.partial(
        pl.kernel,
        out_type=jax.ShapeDtypeStruct((NROW, N_EMBD), jnp.float32),
        mesh=mesh,
        compiler_params=pltpu.CompilerParams(use_tc_tiling_on_sc=False),
        scratch_types=[
            pltpu.VMEM((NCHUNK, C), jnp.int32),
            pltpu.VMEM((C, N_EMBD), jnp.float32),
            pltpu.VMEM((C, N_EMBD), jnp.float32),
            pltpu.SemaphoreType.DMA,
            pltpu.SemaphoreType.DMA,
            pltpu.SemaphoreType.DMA,
            pltpu.SemaphoreType.DMA,
        ],
    )(_sc_body)


def _sc_body(idx_hbm, tok_hbm, out_hbm, idx_v, buf_a, buf_b, gs_a, gs_b, ss_a, ss_b):
    wid = lax.axis_index("s") * NC + lax.axis_index("c")
    base = wid * PER_W
    pltpu.sync_copy(idx_hbm.at[pl.ds(wid * NCHUNK, NCHUNK), :], idx_v)

    bufs = (buf_a, buf_b)
    gsems = (gs_a, gs_b)
    ssems = (ss_a, ss_b)

    def g_start(k, p):
        pltpu.async_copy(tok_hbm.at[idx_v.at[k]], bufs[p], gsems[p])

    def g_wait(k, p):
        pltpu.make_async_copy(tok_hbm.at[idx_v.at[k]], bufs[p], gsems[p]).wait()

    def s_start(k, p):
        pltpu.async_copy(bufs[p], out_hbm.at[pl.ds(base + k * C, C), :], ssems[p])

    def s_wait(k, p):
        pltpu.make_async_copy(
            bufs[p], out_hbm.at[pl.ds(base + k * C, C), :], ssems[p]).wait()

    g_start(0, 0)
    g_start(1, 1)

    def lbody(k2, carry):
        for p in range(2):
            k = k2 * 2 + p
            g_wait(k, p)
            s_start(k, p)
            s_wait(k, p)

            @pl.when(k2 < NCHUNK // 2 - 1)
            def _():
                g_start(k + 2, p)
        return carry

    lax.fori_loop(0, NCHUNK // 2, lbody, 0)


def _proj_body(g_ref, pos_ref, w_ref, b_ref, out_ref):
    for t in range(T):
        x = g_ref[:, t * N_EMBD:(t + 1) * N_EMBD] + pos_ref[pl.ds(t, 1), :]
        y = lax.dot_general(w_ref[...], x.astype(jnp.bfloat16),
                            (((1,), (1,)), ((), ())),
                            preferred_element_type=jnp.float32)
        out_ref[t] = y + b_ref[...]


def _project(g2, pos_table, w_bf, b_col):
    return pl.pallas_call(
        _proj_body,
        grid=(NBLK,),
        in_specs=[
            pl.BlockSpec((BBLK, T * N_EMBD), lambda k: (k, 0)),
            pl.BlockSpec((T, N_EMBD), lambda k: (0, 0)),
            pl.BlockSpec((VOCAB, N_EMBD), lambda k: (0, 0)),
            pl.BlockSpec((VOCAB, 1), lambda k: (0, 0)),
        ],
        out_specs=pl.BlockSpec((T, VOCAB, BBLK), lambda k: (0, 0, k)),
        out_shape=jax.ShapeDtypeStruct((T, VOCAB, BATCH), jnp.float32),
    )(g2, pos_table, w_bf, b_col)


def kernel(idx, tok_table, pos_table, W, b):
    g = _make_sc_gather()(idx.reshape(NROW // C, C), tok_table)
    g2 = g.reshape(BATCH, T * N_EMBD)
    out3 = _project(g2, pos_table, W.astype(jnp.bfloat16), b.reshape(VOCAB, 1))
    return jnp.transpose(out3, (2, 0, 1))
